# baseline (speedup 1.0000x reference)
import functools

import jax
import jax.numpy as jnp
from jax import lax
from jax.experimental import pallas as pl
from jax.experimental.pallas import tpu as pltpu
from jax.experimental.pallas import tpu_sc as plsc

_EPS = 1e-5
_CH = 128


def _sc_gather(face_pad, idx2, E, N, NC, NS):
    NW = NC * NS
    NCH = idx2.shape[0]
    MAIN = (NCH // NW) & ~1
    REM = NCH - MAIN * NW
    mesh = plsc.VectorSubcoreMesh(core_axis_name="c", subcore_axis_name="s")

    @functools.partial(
        pl.kernel,
        out_type=jax.ShapeDtypeStruct((E, 128), jnp.float32),
        mesh=mesh,
        scratch_types=[
            pltpu.VMEM((_CH,), jnp.int32),
            pltpu.VMEM((_CH,), jnp.int32),
            pltpu.VMEM((_CH, 128), jnp.float32),
            pltpu.VMEM((_CH, 128), jnp.float32),
            pltpu.SemaphoreType.DMA,
            pltpu.SemaphoreType.DMA,
            pltpu.SemaphoreType.DMA,
            pltpu.SemaphoreType.DMA,
        ],
    )
    def k(face_hbm, idx2_hbm, g_hbm, idx0, idx1, rows0, rows1, gs0, gs1, ws0, ws1):
        cid = lax.axis_index("c")
        sid = lax.axis_index("s")
        wid = sid * NC + cid
        base = wid * MAIN
        idxs, rows, gss, wss = (idx0, idx1), (rows0, rows1), (gs0, gs1), (ws0, ws1)
        for b in range(2):
            pltpu.sync_copy(idx2_hbm.at[base + b], idxs[b])
            pltpu.async_copy(face_hbm.at[idxs[b]], rows[b], gss[b])

        @pl.loop(0, MAIN // 2)
        def _(t):
            for b in range(2):
                g = 2 * t + b
                dst = g_hbm.at[pl.ds((base + g) * _CH, _CH)]
                pltpu.make_async_copy(face_hbm.at[idxs[b]], rows[b], gss[b]).wait()
                pltpu.async_copy(rows[b], dst, wss[b])
                pltpu.sync_copy(idx2_hbm.at[base + g + 2], idxs[b])
                pltpu.make_async_copy(rows[b], dst, wss[b]).wait()

                @pl.when(g + 2 < MAIN)
                def _():
                    pltpu.async_copy(face_hbm.at[idxs[b]], rows[b], gss[b])

        @pl.when(wid < REM)
        def _():
            ch = NW * MAIN + wid
            pltpu.sync_copy(idx2_hbm.at[ch], idx0)
            pltpu.async_copy(face_hbm.at[idx0], rows0, gs0).wait()
            pltpu.sync_copy(rows0, g_hbm.at[pl.ds(ch * _CH, _CH)])

    return k(face_pad, idx2)


def _sc_scatter(z, idx2, a2, c2, zeros, E, N, NC, NS):
    NW = NC * NS
    NCH = idx2.shape[0]
    MAIN = (NCH // NW) & ~1
    REM = NCH - MAIN * NW
    STRIPE = zeros.shape[0]
    NP = NS * STRIPE
    mesh = plsc.VectorSubcoreMesh(core_axis_name="c", subcore_axis_name="s")

    @functools.partial(
        pl.kernel,
        out_type=jax.ShapeDtypeStruct((NC * NP, 128), jnp.float32),
        mesh=mesh,
        scratch_types=[
            pltpu.VMEM((_CH,), jnp.int32),
            pltpu.VMEM((_CH,), jnp.int32),
            pltpu.VMEM((_CH, 128), jnp.float32),
            pltpu.VMEM((_CH, 128), jnp.float32),
            pltpu.VMEM((128,), jnp.float32),
            pltpu.VMEM((128,), jnp.float32),
            pltpu.VMEM_SHARED((NP, 128), jnp.float32),
            pltpu.SemaphoreType.DMA,
            pltpu.SemaphoreType.DMA,
        ],
    )
    def k(z_hbm, idx2_hbm, a_hbm, c_hbm, zeros_hbm, out_hbm,
          idx0, idx1, zb0, zb1, abuf, cbuf, agg_sh, zs0, zs1):
        cid = lax.axis_index("c")
        sid = lax.axis_index("s")
        wid = sid * NC + cid
        pltpu.sync_copy(zeros_hbm, agg_sh.at[pl.ds(sid * STRIPE, STRIPE)])
        pltpu.sync_copy(a_hbm, abuf)
        pltpu.sync_copy(c_hbm, cbuf)
        plsc.subcore_barrier()
        av = [abuf[pl.ds(16 * q, 16)] for q in range(8)]
        cv = [cbuf[pl.ds(16 * q, 16)] for q in range(8)]
        idxs, zbufs, zss = (idx0, idx1), (zb0, zb1), (zs0, zs1)
        base = wid * MAIN

        def relu_affine(zbuf):
            def row(r, inner):
                for q in range(8):
                    sl = pl.ds(16 * q, 16)
                    zbuf[r, sl] = jnp.maximum(zbuf[r, sl] * av[q] + cv[q], 0.0)
                return inner

            lax.fori_loop(0, _CH, row, 0)

        for b in range(2):
            pltpu.sync_copy(idx2_hbm.at[base + b], idxs[b])
            pltpu.async_copy(z_hbm.at[pl.ds((base + b) * _CH, _CH)], zbufs[b], zss[b])

        @pl.loop(0, MAIN // 2)
        def _(t):
            for b in range(2):
                g = 2 * t + b
                pltpu.make_async_copy(z_hbm.at[pl.ds((base + g) * _CH, _CH)],
                                      zbufs[b], zss[b]).wait()
                relu_affine(zbufs[b])
                pltpu.sync_copy(zbufs[b], agg_sh.at[idxs[b]], add=True)
                pltpu.sync_copy(idx2_hbm.at[base + g + 2], idxs[b])

                @pl.when(g + 2 < MAIN)
                def _():
                    pltpu.async_copy(z_hbm.at[pl.ds((base + g + 2) * _CH, _CH)],
                                     zbufs[b], zss[b])

        @pl.when(wid < REM)
        def _():
            ch = NW * MAIN + wid
            pltpu.sync_copy(idx2_hbm.at[ch], idx0)
            pltpu.sync_copy(z_hbm.at[pl.ds(ch * _CH, _CH)], zb0)
            relu_affine(zb0)
            pltpu.sync_copy(zb0, agg_sh.at[idx0], add=True)

        plsc.subcore_barrier()
        pltpu.sync_copy(agg_sh.at[pl.ds(sid * STRIPE, STRIPE)],
                        out_hbm.at[pl.ds(cid * NP + sid * STRIPE, STRIPE)])

    return k(z, idx2, a2, c2, zeros)


def _tc_edge_stats(e, BE):
    E = e.shape[0]
    nsteps = E // BE

    def body(e_ref, see_ref, see_acc):
        i = pl.program_id(0)

        @pl.when(i == 0)
        def _():
            see_acc[...] = jnp.zeros_like(see_acc)

        eb = e_ref[...]
        see_acc[...] += lax.dot_general(eb, eb, (((0,), (0,)), ((), ())),
                                        preferred_element_type=jnp.float32)

        @pl.when(i == nsteps - 1)
        def _():
            see_ref[...] = see_acc[...]

    return pl.pallas_call(
        body,
        grid=(nsteps,),
        in_specs=[pl.BlockSpec((BE, 64), lambda i: (i, 0))],
        out_specs=pl.BlockSpec((64, 64), lambda i: (0, 0)),
        out_shape=jax.ShapeDtypeStruct((64, 64), jnp.float32),
        scratch_shapes=[pltpu.VMEM((64, 64), jnp.float32)],
        compiler_params=pltpu.CompilerParams(dimension_semantics=("arbitrary",)),
    )(e)


def _tc_stats_fold(g, e, see, W1, g1, b1, BE):
    E = g.shape[0]
    nsteps = E // BE
    K = W1.shape[0]
    M = W1.shape[1]

    def body(g_ref, e_ref, see_ref, w1_ref, g1_ref, b1_ref,
             w1f_ref, c1_ref, a_acc, xsum):
        i = pl.program_id(0)

        @pl.when(i == 0)
        def _():
            a_acc[...] = jnp.zeros_like(a_acc)
            xsum[...] = jnp.zeros_like(xsum)

        xg = g_ref[...][:, :64]
        x = jnp.concatenate([xg, e_ref[...]], axis=1)
        a_acc[...] += lax.dot_general(xg, x, (((0,), (0,)), ((), ())),
                                      preferred_element_type=jnp.float32)
        xsum[...] += jnp.sum(x, axis=0, keepdims=True)

        @pl.when(i == nsteps - 1)
        def _():
            w1 = w1_ref[...]
            wg = w1[:64, :]
            we = w1[64:, :]
            m1 = jnp.dot(xsum[...], w1, preferred_element_type=jnp.float32) / E
            ggg = a_acc[...][:, :64]
            gge = a_acc[...][:, 64:]
            t = (jnp.sum(wg * jnp.dot(ggg, wg, preferred_element_type=jnp.float32),
                         axis=0, keepdims=True)
                 + 2.0 * jnp.sum(wg * jnp.dot(gge, we, preferred_element_type=jnp.float32),
                                 axis=0, keepdims=True)
                 + jnp.sum(we * jnp.dot(see_ref[...], we, preferred_element_type=jnp.float32),
                           axis=0, keepdims=True))
            v1 = t / E - m1 * m1
            a1 = g1_ref[...] * lax.rsqrt(v1 + _EPS)
            c1_ref[...] = b1_ref[...] - m1 * a1
            w1f_ref[...] = w1 * a1

    return pl.pallas_call(
        body,
        grid=(nsteps,),
        in_specs=[
            pl.BlockSpec((BE, 128), lambda i: (i, 0)),
            pl.BlockSpec((BE, 64), lambda i: (i, 0)),
            pl.BlockSpec((64, 64), lambda i: (0, 0)),
            pl.BlockSpec((K, M), lambda i: (0, 0)),
            pl.BlockSpec((1, M), lambda i: (0, 0)),
            pl.BlockSpec((1, M), lambda i: (0, 0)),
        ],
        out_specs=[
            pl.BlockSpec((K, M), lambda i: (0, 0)),
            pl.BlockSpec((1, M), lambda i: (0, 0)),
        ],
        out_shape=[
            jax.ShapeDtypeStruct((K, M), jnp.float32),
            jax.ShapeDtypeStruct((1, M), jnp.float32),
        ],
        scratch_shapes=[
            pltpu.VMEM((64, K), jnp.float32),
            pltpu.VMEM((1, K), jnp.float32),
        ],
        compiler_params=pltpu.CompilerParams(dimension_semantics=("arbitrary",)),
    )(g, e, see, W1, g1.reshape(1, -1), b1.reshape(1, -1))


def _tc_mlp1(g, e, w1f, c1, W2, g2, b2, BE):
    E = g.shape[0]
    nsteps = E // BE
    K = w1f.shape[0]
    M = w1f.shape[1]
    P = W2.shape[1]

    def body(g_ref, e_ref, w1f_ref, c1_ref, w2_ref, g2_ref, b2_ref,
             z_ref, a2_ref, c2_ref, zsum, zsq):
        i = pl.program_id(0)

        @pl.when(i == 0)
        def _():
            zsum[...] = jnp.zeros_like(zsum)
            zsq[...] = jnp.zeros_like(zsq)

        x = jnp.concatenate([g_ref[...][:, :64], e_ref[...]], axis=1)
        h = jnp.maximum(jnp.dot(x, w1f_ref[...], preferred_element_type=jnp.float32)
                        + c1_ref[...], 0.0)
        z = jnp.dot(h, w2_ref[...], preferred_element_type=jnp.float32)
        z_ref[...] = z
        zsum[...] += jnp.sum(z, axis=0, keepdims=True)
        zsq[...] += jnp.sum(z * z, axis=0, keepdims=True)

        @pl.when(i == nsteps - 1)
        def _():
            m2 = zsum[...] / E
            v2 = zsq[...] / E - m2 * m2
            a2 = g2_ref[...] * lax.rsqrt(v2 + _EPS)
            a2_ref[...] = a2
            c2_ref[...] = b2_ref[...] - m2 * a2

    return pl.pallas_call(
        body,
        grid=(nsteps,),
        in_specs=[
            pl.BlockSpec((BE, 128), lambda i: (i, 0)),
            pl.BlockSpec((BE, 64), lambda i: (i, 0)),
            pl.BlockSpec((K, M), lambda i: (0, 0)),
            pl.BlockSpec((1, M), lambda i: (0, 0)),
            pl.BlockSpec((M, P), lambda i: (0, 0)),
            pl.BlockSpec((1, P), lambda i: (0, 0)),
            pl.BlockSpec((1, P), lambda i: (0, 0)),
        ],
        out_specs=[
            pl.BlockSpec((BE, P), lambda i: (i, 0)),
            pl.BlockSpec((1, P), lambda i: (0, 0)),
            pl.BlockSpec((1, P), lambda i: (0, 0)),
        ],
        out_shape=[
            jax.ShapeDtypeStruct((E, P), jnp.float32),
            jax.ShapeDtypeStruct((1, P), jnp.float32),
            jax.ShapeDtypeStruct((1, P), jnp.float32),
        ],
        scratch_shapes=[
            pltpu.VMEM((1, P), jnp.float32),
            pltpu.VMEM((1, P), jnp.float32),
        ],
        compiler_params=pltpu.CompilerParams(dimension_semantics=("arbitrary",)),
    )(g, e, w1f, c1, W2, g2.reshape(1, -1), b2.reshape(1, -1))


def _tc_agg_stats_fold(p0, p1, W3, g3, b3, BN):
    N = p0.shape[0]
    nsteps = N // BN
    K = W3.shape[0]
    M = W3.shape[1]

    def body(p0_ref, p1_ref, w3_ref, g3_ref, b3_ref,
             agg_ref, w3f_ref, c3_ref, ata, asum):
        i = pl.program_id(0)

        @pl.when(i == 0)
        def _():
            ata[...] = jnp.zeros_like(ata)
            asum[...] = jnp.zeros_like(asum)

        a = p0_ref[...] + p1_ref[...]
        agg_ref[...] = a
        ata[...] += lax.dot_general(a, a, (((0,), (0,)), ((), ())),
                                    preferred_element_type=jnp.float32)
        asum[...] += jnp.sum(a, axis=0, keepdims=True)

        @pl.when(i == nsteps - 1)
        def _():
            w3 = w3_ref[...]
            m3 = jnp.dot(asum[...], w3, preferred_element_type=jnp.float32) / N
            ex2 = jnp.sum(w3 * jnp.dot(ata[...], w3, preferred_element_type=jnp.float32),
                          axis=0, keepdims=True) / N
            v3 = ex2 - m3 * m3
            a3 = g3_ref[...] * lax.rsqrt(v3 + _EPS)
            c3_ref[...] = b3_ref[...] - m3 * a3
            w3f_ref[...] = w3 * a3

    return pl.pallas_call(
        body,
        grid=(nsteps,),
        in_specs=[
            pl.BlockSpec((BN, K), lambda i: (i, 0)),
            pl.BlockSpec((BN, K), lambda i: (i, 0)),
            pl.BlockSpec((K, M), lambda i: (0, 0)),
            pl.BlockSpec((1, M), lambda i: (0, 0)),
            pl.BlockSpec((1, M), lambda i: (0, 0)),
        ],
        out_specs=[
            pl.BlockSpec((BN, K), lambda i: (i, 0)),
            pl.BlockSpec((K, M), lambda i: (0, 0)),
            pl.BlockSpec((1, M), lambda i: (0, 0)),
        ],
        out_shape=[
            jax.ShapeDtypeStruct((N, K), jnp.float32),
            jax.ShapeDtypeStruct((K, M), jnp.float32),
            jax.ShapeDtypeStruct((1, M), jnp.float32),
        ],
        scratch_shapes=[
            pltpu.VMEM((K, K), jnp.float32),
            pltpu.VMEM((1, K), jnp.float32),
        ],
        compiler_params=pltpu.CompilerParams(dimension_semantics=("arbitrary",)),
    )(p0, p1, W3, g3.reshape(1, -1), b3.reshape(1, -1))


def _tc_mlp2_mid(agg, w3f, c3, W4, g4, b4, BN):
    N = agg.shape[0]
    nsteps = N // BN
    K = agg.shape[1]
    M = w3f.shape[1]

    def body(agg_ref, w3f_ref, c3_ref, w4_ref, g4_ref, b4_ref,
             z4_ref, a4_ref, c4_ref, zsum, zsq):
        i = pl.program_id(0)

        @pl.when(i == 0)
        def _():
            zsum[...] = jnp.zeros_like(zsum)
            zsq[...] = jnp.zeros_like(zsq)

        h = jnp.maximum(jnp.dot(agg_ref[...], w3f_ref[...],
                                preferred_element_type=jnp.float32) + c3_ref[...], 0.0)
        z = jnp.dot(h, w4_ref[...], preferred_element_type=jnp.float32)
        z4_ref[...] = z
        zsum[...] += jnp.sum(z, axis=0, keepdims=True)
        zsq[...] += jnp.sum(z * z, axis=0, keepdims=True)

        @pl.when(i == nsteps - 1)
        def _():
            m4 = zsum[...] / N
            v4 = zsq[...] / N - m4 * m4
            a4 = g4_ref[...] * lax.rsqrt(v4 + _EPS)
            a4_ref[...] = a4
            c4_ref[...] = b4_ref[...] - m4 * a4

    return pl.pallas_call(
        body,
        grid=(nsteps,),
        in_specs=[
            pl.BlockSpec((BN, K), lambda i: (i, 0)),
            pl.BlockSpec((K, M), lambda i: (0, 0)),
            pl.BlockSpec((1, M), lambda i: (0, 0)),
            pl.BlockSpec((M, M), lambda i: (0, 0)),
            pl.BlockSpec((1, M), lambda i: (0, 0)),
            pl.BlockSpec((1, M), lambda i: (0, 0)),
        ],
        out_specs=[
            pl.BlockSpec((BN, M), lambda i: (i, 0)),
            pl.BlockSpec((1, M), lambda i: (0, 0)),
            pl.BlockSpec((1, M), lambda i: (0, 0)),
        ],
        out_shape=[
            jax.ShapeDtypeStruct((N, M), jnp.float32),
            jax.ShapeDtypeStruct((1, M), jnp.float32),
            jax.ShapeDtypeStruct((1, M), jnp.float32),
        ],
        scratch_shapes=[
            pltpu.VMEM((1, M), jnp.float32),
            pltpu.VMEM((1, M), jnp.float32),
        ],
        compiler_params=pltpu.CompilerParams(dimension_semantics=("arbitrary",)),
    )(agg, w3f, c3, W4, g4.reshape(1, -1), b4.reshape(1, -1))


def _tc_mlp2_out(z4, a4, c4, W5, b5, BN):
    N = z4.shape[0]
    nsteps = N // BN
    M = z4.shape[1]
    P = W5.shape[1]

    def body(z4_ref, a4_ref, c4_ref, w5_ref, b5_ref, out_ref):
        y = jnp.maximum(z4_ref[...] * a4_ref[...] + c4_ref[...], 0.0)
        out_ref[...] = jnp.dot(y, w5_ref[...],
                               preferred_element_type=jnp.float32) + b5_ref[...]

    return pl.pallas_call(
        body,
        grid=(nsteps,),
        in_specs=[
            pl.BlockSpec((BN, M), lambda i: (i, 0)),
            pl.BlockSpec((1, M), lambda i: (0, 0)),
            pl.BlockSpec((1, M), lambda i: (0, 0)),
            pl.BlockSpec((M, P), lambda i: (0, 0)),
            pl.BlockSpec((1, P), lambda i: (0, 0)),
        ],
        out_specs=pl.BlockSpec((BN, P), lambda i: (i, 0)),
        out_shape=jax.ShapeDtypeStruct((N, P), jnp.float32),
        compiler_params=pltpu.CompilerParams(dimension_semantics=("arbitrary",)),
    )(z4, a4, c4, W5, b5.reshape(1, -1))


def kernel(face_rep, edge_rep, domain_indicator, W1, g1, b1, W2, g2, b2,
           W3, g3, b3, W4, g4, b4, W5, b5):
    E = edge_rep.shape[0]
    N = face_rep.shape[0]
    info = plsc.get_sparse_core_info()
    NC, NS = info.num_cores, info.num_subcores

    idx2 = domain_indicator.astype(jnp.int32).reshape(E // _CH, _CH)

    face_pad = jnp.pad(face_rep, ((0, 0), (0, 64)))
    gathered = _sc_gather(face_pad, idx2, E, N, NC, NS)

    BE = 3200
    see = _tc_edge_stats(edge_rep, BE)
    w1f, c1 = _tc_stats_fold(gathered, edge_rep, see, W1, g1, b1, BE)
    z, a2, c2 = _tc_mlp1(gathered, edge_rep, w1f, c1, W2, g2, b2, BE)

    STRIPE = ((N + NS - 1) // NS + 7) // 8 * 8
    NP = NS * STRIPE
    zeros = jnp.zeros((STRIPE, 128), jnp.float32)
    parts = _sc_scatter(z, idx2, a2.reshape(-1), c2.reshape(-1), zeros, E, N, NC, NS)
    p0, p1 = parts[:N], parts[NP:NP + N]

    BN = 2000
    agg, w3f, c3 = _tc_agg_stats_fold(p0, p1, W3, g3, b3, BN)
    z4, a4, c4 = _tc_mlp2_mid(agg, w3f, c3, W4, g4, b4, BN)
    out = _tc_mlp2_out(z4, a4, c4, W5, b5, BN)
    return out


# speedup vs baseline: 1.1363x; 1.1363x over previous
import functools

import jax
import jax.numpy as jnp
from jax import lax
from jax.experimental import pallas as pl
from jax.experimental.pallas import tpu as pltpu
from jax.experimental.pallas import tpu_sc as plsc

_EPS = 1e-5
_CH = 128


def _sc_gather(face_pad, idx2, E, N, NC, NS):
    NW = NC * NS
    NCH = idx2.shape[0]
    MAIN = (NCH // NW) & ~1
    REM = NCH - MAIN * NW
    mesh = plsc.VectorSubcoreMesh(core_axis_name="c", subcore_axis_name="s")

    @functools.partial(
        pl.kernel,
        out_type=jax.ShapeDtypeStruct((E, 128), jnp.float32),
        mesh=mesh,
        scratch_types=[
            pltpu.VMEM((_CH,), jnp.int32),
            pltpu.VMEM((_CH,), jnp.int32),
            pltpu.VMEM((_CH, 128), jnp.float32),
            pltpu.VMEM((_CH, 128), jnp.float32),
            pltpu.SemaphoreType.DMA,
            pltpu.SemaphoreType.DMA,
            pltpu.SemaphoreType.DMA,
            pltpu.SemaphoreType.DMA,
        ],
    )
    def k(face_hbm, idx2_hbm, g_hbm, idx0, idx1, rows0, rows1, gs0, gs1, ws0, ws1):
        cid = lax.axis_index("c")
        sid = lax.axis_index("s")
        wid = sid * NC + cid
        base = wid * MAIN
        idxs, rows, gss, wss = (idx0, idx1), (rows0, rows1), (gs0, gs1), (ws0, ws1)
        for b in range(2):
            pltpu.sync_copy(idx2_hbm.at[base + b], idxs[b])
            pltpu.async_copy(face_hbm.at[idxs[b]], rows[b], gss[b])

        @pl.loop(0, MAIN // 2)
        def _(t):
            for b in range(2):
                g = 2 * t + b
                dst = g_hbm.at[pl.ds((base + g) * _CH, _CH)]
                pltpu.make_async_copy(face_hbm.at[idxs[b]], rows[b], gss[b]).wait()
                pltpu.async_copy(rows[b], dst, wss[b])
                pltpu.sync_copy(idx2_hbm.at[base + g + 2], idxs[b])
                pltpu.make_async_copy(rows[b], dst, wss[b]).wait()

                @pl.when(g + 2 < MAIN)
                def _():
                    pltpu.async_copy(face_hbm.at[idxs[b]], rows[b], gss[b])

        @pl.when(wid < REM)
        def _():
            ch = NW * MAIN + wid
            pltpu.sync_copy(idx2_hbm.at[ch], idx0)
            pltpu.async_copy(face_hbm.at[idx0], rows0, gs0).wait()
            pltpu.sync_copy(rows0, g_hbm.at[pl.ds(ch * _CH, _CH)])

    return k(face_pad, idx2)


def _sc_scatter(z, idx2, a2, c2, zeros, E, N, NC, NS):
    NW = NC * NS
    NCH = idx2.shape[0]
    MAIN = (NCH // NW) & ~1
    REM = NCH - MAIN * NW
    STRIPE = zeros.shape[0]
    NP = NS * STRIPE
    mesh = plsc.VectorSubcoreMesh(core_axis_name="c", subcore_axis_name="s")

    @functools.partial(
        pl.kernel,
        out_type=jax.ShapeDtypeStruct((NC * NP, 128), jnp.float32),
        mesh=mesh,
        scratch_types=[
            pltpu.VMEM((_CH,), jnp.int32),
            pltpu.VMEM((_CH,), jnp.int32),
            pltpu.VMEM((_CH, 128), jnp.float32),
            pltpu.VMEM((_CH, 128), jnp.float32),
            pltpu.VMEM((128,), jnp.float32),
            pltpu.VMEM((128,), jnp.float32),
            pltpu.VMEM_SHARED((NP, 128), jnp.float32),
            pltpu.SemaphoreType.DMA,
            pltpu.SemaphoreType.DMA,
        ],
    )
    def k(z_hbm, idx2_hbm, a_hbm, c_hbm, zeros_hbm, out_hbm,
          idx0, idx1, zb0, zb1, abuf, cbuf, agg_sh, zs0, zs1):
        cid = lax.axis_index("c")
        sid = lax.axis_index("s")
        wid = sid * NC + cid
        pltpu.sync_copy(zeros_hbm, agg_sh.at[pl.ds(sid * STRIPE, STRIPE)])
        pltpu.sync_copy(a_hbm, abuf)
        pltpu.sync_copy(c_hbm, cbuf)
        plsc.subcore_barrier()
        av = [abuf[pl.ds(16 * q, 16)] for q in range(8)]
        cv = [cbuf[pl.ds(16 * q, 16)] for q in range(8)]
        idxs, zbufs, zss = (idx0, idx1), (zb0, zb1), (zs0, zs1)
        base = wid * MAIN

        def relu_affine(zbuf):
            def row(r, inner):
                for q in range(8):
                    sl = pl.ds(16 * q, 16)
                    zbuf[r, sl] = jnp.maximum(zbuf[r, sl] * av[q] + cv[q], 0.0)
                return inner

            lax.fori_loop(0, _CH, row, 0)

        for b in range(2):
            pltpu.sync_copy(idx2_hbm.at[base + b], idxs[b])
            pltpu.async_copy(z_hbm.at[pl.ds((base + b) * _CH, _CH)], zbufs[b], zss[b])

        @pl.loop(0, MAIN // 2)
        def _(t):
            for b in range(2):
                g = 2 * t + b
                pltpu.make_async_copy(z_hbm.at[pl.ds((base + g) * _CH, _CH)],
                                      zbufs[b], zss[b]).wait()
                relu_affine(zbufs[b])
                pltpu.sync_copy(zbufs[b], agg_sh.at[idxs[b]], add=True)
                pltpu.sync_copy(idx2_hbm.at[base + g + 2], idxs[b])

                @pl.when(g + 2 < MAIN)
                def _():
                    pltpu.async_copy(z_hbm.at[pl.ds((base + g + 2) * _CH, _CH)],
                                     zbufs[b], zss[b])

        @pl.when(wid < REM)
        def _():
            ch = NW * MAIN + wid
            pltpu.sync_copy(idx2_hbm.at[ch], idx0)
            pltpu.sync_copy(z_hbm.at[pl.ds(ch * _CH, _CH)], zb0)
            relu_affine(zb0)
            pltpu.sync_copy(zb0, agg_sh.at[idx0], add=True)

        plsc.subcore_barrier()
        pltpu.sync_copy(agg_sh.at[pl.ds(sid * STRIPE, STRIPE)],
                        out_hbm.at[pl.ds(cid * NP + sid * STRIPE, STRIPE)])

    return k(z, idx2, a2, c2, zeros)


def _tc_stats_fold(g, e, W1, g1, b1, BE):
    E = g.shape[0]
    nsteps = E // BE
    K = W1.shape[0]
    M = W1.shape[1]

    def body(g_ref, e_ref, w1_ref, g1_ref, b1_ref, w1f_ref, c1_ref, xtx, xsum):
        i = pl.program_id(0)

        @pl.when(i == 0)
        def _():
            xtx[...] = jnp.zeros_like(xtx)
            xsum[...] = jnp.zeros_like(xsum)

        x = jnp.concatenate([g_ref[...][:, :64], e_ref[...]], axis=1)
        xtx[...] += lax.dot_general(x, x, (((0,), (0,)), ((), ())),
                                    preferred_element_type=jnp.float32)
        xsum[...] += jnp.sum(x, axis=0, keepdims=True)

        @pl.when(i == nsteps - 1)
        def _():
            w1 = w1_ref[...]
            m1 = jnp.dot(xsum[...], w1, preferred_element_type=jnp.float32) / E
            ex2 = jnp.sum(w1 * jnp.dot(xtx[...], w1, preferred_element_type=jnp.float32),
                          axis=0, keepdims=True) / E
            v1 = ex2 - m1 * m1
            a1 = g1_ref[...] * lax.rsqrt(v1 + _EPS)
            c1_ref[...] = b1_ref[...] - m1 * a1
            w1f_ref[...] = w1 * a1

    return pl.pallas_call(
        body,
        grid=(nsteps,),
        in_specs=[
            pl.BlockSpec((BE, 128), lambda i: (i, 0)),
            pl.BlockSpec((BE, 64), lambda i: (i, 0)),
            pl.BlockSpec((K, M), lambda i: (0, 0)),
            pl.BlockSpec((1, M), lambda i: (0, 0)),
            pl.BlockSpec((1, M), lambda i: (0, 0)),
        ],
        out_specs=[
            pl.BlockSpec((K, M), lambda i: (0, 0)),
            pl.BlockSpec((1, M), lambda i: (0, 0)),
        ],
        out_shape=[
            jax.ShapeDtypeStruct((K, M), jnp.float32),
            jax.ShapeDtypeStruct((1, M), jnp.float32),
        ],
        scratch_shapes=[
            pltpu.VMEM((K, K), jnp.float32),
            pltpu.VMEM((1, K), jnp.float32),
        ],
        compiler_params=pltpu.CompilerParams(dimension_semantics=("arbitrary",)),
    )(g, e, W1, g1.reshape(1, -1), b1.reshape(1, -1))


def _tc_mlp1(g, e, w1f, c1, W2, g2, b2, BE):
    E = g.shape[0]
    nsteps = E // BE
    K = w1f.shape[0]
    M = w1f.shape[1]
    P = W2.shape[1]

    def body(g_ref, e_ref, w1f_ref, c1_ref, w2_ref, g2_ref, b2_ref,
             z_ref, a2_ref, c2_ref, zsum, zsq):
        i = pl.program_id(0)

        @pl.when(i == 0)
        def _():
            zsum[...] = jnp.zeros_like(zsum)
            zsq[...] = jnp.zeros_like(zsq)

        x = jnp.concatenate([g_ref[...][:, :64], e_ref[...]], axis=1)
        h = jnp.maximum(jnp.dot(x, w1f_ref[...], preferred_element_type=jnp.float32)
                        + c1_ref[...], 0.0)
        z = jnp.dot(h, w2_ref[...], preferred_element_type=jnp.float32)
        z_ref[...] = z
        zsum[...] += jnp.sum(z, axis=0, keepdims=True)
        zsq[...] += jnp.sum(z * z, axis=0, keepdims=True)

        @pl.when(i == nsteps - 1)
        def _():
            m2 = zsum[...] / E
            v2 = zsq[...] / E - m2 * m2
            a2 = g2_ref[...] * lax.rsqrt(v2 + _EPS)
            a2_ref[...] = a2
            c2_ref[...] = b2_ref[...] - m2 * a2

    return pl.pallas_call(
        body,
        grid=(nsteps,),
        in_specs=[
            pl.BlockSpec((BE, 128), lambda i: (i, 0)),
            pl.BlockSpec((BE, 64), lambda i: (i, 0)),
            pl.BlockSpec((K, M), lambda i: (0, 0)),
            pl.BlockSpec((1, M), lambda i: (0, 0)),
            pl.BlockSpec((M, P), lambda i: (0, 0)),
            pl.BlockSpec((1, P), lambda i: (0, 0)),
            pl.BlockSpec((1, P), lambda i: (0, 0)),
        ],
        out_specs=[
            pl.BlockSpec((BE, P), lambda i: (i, 0)),
            pl.BlockSpec((1, P), lambda i: (0, 0)),
            pl.BlockSpec((1, P), lambda i: (0, 0)),
        ],
        out_shape=[
            jax.ShapeDtypeStruct((E, P), jnp.float32),
            jax.ShapeDtypeStruct((1, P), jnp.float32),
            jax.ShapeDtypeStruct((1, P), jnp.float32),
        ],
        scratch_shapes=[
            pltpu.VMEM((1, P), jnp.float32),
            pltpu.VMEM((1, P), jnp.float32),
        ],
        compiler_params=pltpu.CompilerParams(dimension_semantics=("arbitrary",)),
    )(g, e, w1f, c1, W2, g2.reshape(1, -1), b2.reshape(1, -1))


def _tc_mlp2_fused(p0, p1, W3, g3, b3, W4, g4, b4, W5, b5, BN):
    N = p0.shape[0]
    ns = N // BN
    K = W3.shape[0]
    M = W3.shape[1]

    def body(p0_ref, p1_ref, w3_ref, g3_ref, b3_ref, w4_ref, g4_ref, b4_ref,
             w5_ref, b5_ref, out_ref,
             ata, asum, w3f, c3, z4s, zsum, zsq, a4, c4):
        i = pl.program_id(0)

        @pl.when(i == 0)
        def _():
            ata[...] = jnp.zeros_like(ata)
            asum[...] = jnp.zeros_like(asum)
            zsum[...] = jnp.zeros_like(zsum)
            zsq[...] = jnp.zeros_like(zsq)

        @pl.when(i < ns)
        def _():
            a = p0_ref[...] + p1_ref[...]
            ata[...] += lax.dot_general(a, a, (((0,), (0,)), ((), ())),
                                        preferred_element_type=jnp.float32)
            asum[...] += jnp.sum(a, axis=0, keepdims=True)

            @pl.when(i == ns - 1)
            def _():
                w3 = w3_ref[...]
                m3 = jnp.dot(asum[...], w3, preferred_element_type=jnp.float32) / N
                ex2 = jnp.sum(w3 * jnp.dot(ata[...], w3,
                                           preferred_element_type=jnp.float32),
                              axis=0, keepdims=True) / N
                v3 = ex2 - m3 * m3
                a3 = g3_ref[...] * lax.rsqrt(v3 + _EPS)
                c3[...] = b3_ref[...] - m3 * a3
                w3f[...] = w3 * a3

        @pl.when(jnp.logical_and(i >= ns, i < 2 * ns))
        def _():
            j = i - ns
            a = p0_ref[...] + p1_ref[...]
            h = jnp.maximum(jnp.dot(a, w3f[...],
                                    preferred_element_type=jnp.float32) + c3[...], 0.0)
            z = jnp.dot(h, w4_ref[...], preferred_element_type=jnp.float32)
            z4s[pl.ds(j * BN, BN), :] = z
            zsum[...] += jnp.sum(z, axis=0, keepdims=True)
            zsq[...] += jnp.sum(z * z, axis=0, keepdims=True)

            @pl.when(i == 2 * ns - 1)
            def _():
                m4 = zsum[...] / N
                v4 = zsq[...] / N - m4 * m4
                a4[...] = g4_ref[...] * lax.rsqrt(v4 + _EPS)
                c4[...] = b4_ref[...] - m4 * a4[...]

        @pl.when(i >= 2 * ns)
        def _():
            j = i - 2 * ns
            y = jnp.maximum(z4s[pl.ds(j * BN, BN), :] * a4[...] + c4[...], 0.0)
            out_ref[...] = jnp.dot(y, w5_ref[...],
                                   preferred_element_type=jnp.float32) + b5_ref[...]

    def pmap(i):
        return (lax.rem(i, ns), 0)

    def omap(i):
        return (jnp.where(i >= 2 * ns, i - 2 * ns, 0), 0)

    return pl.pallas_call(
        body,
        grid=(3 * ns,),
        in_specs=[
            pl.BlockSpec((BN, K), pmap),
            pl.BlockSpec((BN, K), pmap),
            pl.BlockSpec((K, M), lambda i: (0, 0)),
            pl.BlockSpec((1, M), lambda i: (0, 0)),
            pl.BlockSpec((1, M), lambda i: (0, 0)),
            pl.BlockSpec((M, M), lambda i: (0, 0)),
            pl.BlockSpec((1, M), lambda i: (0, 0)),
            pl.BlockSpec((1, M), lambda i: (0, 0)),
            pl.BlockSpec((M, K), lambda i: (0, 0)),
            pl.BlockSpec((1, K), lambda i: (0, 0)),
        ],
        out_specs=pl.BlockSpec((BN, K), omap),
        out_shape=jax.ShapeDtypeStruct((N, K), jnp.float32),
        scratch_shapes=[
            pltpu.VMEM((K, K), jnp.float32),
            pltpu.VMEM((1, K), jnp.float32),
            pltpu.VMEM((K, M), jnp.float32),
            pltpu.VMEM((1, M), jnp.float32),
            pltpu.VMEM((N, M), jnp.float32),
            pltpu.VMEM((1, M), jnp.float32),
            pltpu.VMEM((1, M), jnp.float32),
            pltpu.VMEM((1, M), jnp.float32),
            pltpu.VMEM((1, M), jnp.float32),
        ],
        compiler_params=pltpu.CompilerParams(dimension_semantics=("arbitrary",)),
    )(p0, p1, W3, g3.reshape(1, -1), b3.reshape(1, -1), W4,
      g4.reshape(1, -1), b4.reshape(1, -1), W5, b5.reshape(1, -1))


def kernel(face_rep, edge_rep, domain_indicator, W1, g1, b1, W2, g2, b2,
           W3, g3, b3, W4, g4, b4, W5, b5):
    E = edge_rep.shape[0]
    N = face_rep.shape[0]
    info = plsc.get_sparse_core_info()
    NC, NS = info.num_cores, info.num_subcores

    idx2 = domain_indicator.astype(jnp.int32).reshape(E // _CH, _CH)

    face_pad = jnp.pad(face_rep, ((0, 0), (0, 64)))
    gathered = _sc_gather(face_pad, idx2, E, N, NC, NS)

    BE = 3200
    w1f, c1 = _tc_stats_fold(gathered, edge_rep, W1, g1, b1, BE)
    z, a2, c2 = _tc_mlp1(gathered, edge_rep, w1f, c1, W2, g2, b2, BE)

    STRIPE = ((N + NS - 1) // NS + 7) // 8 * 8
    NP = NS * STRIPE
    zeros = jnp.zeros((STRIPE, 128), jnp.float32)
    parts = _sc_scatter(z, idx2, a2.reshape(-1), c2.reshape(-1), zeros, E, N, NC, NS)
    p0, p1 = parts[:N], parts[NP:NP + N]

    BN = 2000
    out = _tc_mlp2_fused(p0, p1, W3, g3, b3, W4, g4, b4, W5, b5, BN)
    return out
